# baseline (speedup 1.0000x reference)
import functools

import jax
import jax.numpy as jnp
from jax import lax
from jax.experimental import pallas as pl
from jax.experimental.pallas import tpu as pltpu
from jax.experimental.pallas import tpu_sc as plsc

NW = 32
HOP1 = 128
G = 8


def _make_gather(B, K):
    TPW = B // NW
    H = K // 2
    mesh = plsc.VectorSubcoreMesh(core_axis_name="c", subcore_axis_name="s")

    @functools.partial(
        pl.kernel,
        mesh=mesh,
        out_type=[
            jax.ShapeDtypeStruct((B * H, 128), jnp.float32),
            jax.ShapeDtypeStruct((B * H, 128), jnp.float32),
        ],
        scratch_types=[
            pltpu.VMEM((TPW,), jnp.int32),
            pltpu.VMEM((HOP1, 128), jnp.int32),
            pltpu.VMEM((G * K, 128), jnp.float32),
            pltpu.VMEM((G * K, 128), jnp.float32),
            pltpu.VMEM((G * H, 128), jnp.float32),
            pltpu.VMEM((G * H, 128), jnp.float32),
            pltpu.SemaphoreType.DMA,
            pltpu.SemaphoreType.DMA,
            pltpu.SemaphoreType.DMA,
            pltpu.SemaphoreType.DMA,
        ],
    )
    def gather_k(uids, iids, pidx, ptab, gu_out, gi_out,
                 tok_v, neighs_v, ring0, ring1, cbuf0, cbuf1,
                 sem_h, sem_g, sem_w0, sem_w1):
        wid = lax.axis_index("s") * 2 + lax.axis_index("c")
        base = wid * TPW

        def do_side(ids_hbm, col_off, poff, out_hbm):
            pltpu.sync_copy(ids_hbm.at[pl.ds(base, TPW)], tok_v)

            def fire(ring, toff):
                return [
                    pltpu.async_copy(
                        ptab.at[neighs_v.at[toff + t, pl.ds(col_off, K)]],
                        ring.at[pl.ds(t * K, K)], sem_g)
                    for t in range(G)
                ]

            def compact(ring, cbuf):
                def comp(q, c):
                    t = q // H
                    i = q - t * H
                    r0 = t * K + i
                    r1 = r0 + H
                    for h in range(4):
                        cbuf[q, pl.ds(h * 16, 16)] = \
                            ring[r0, pl.ds(poff + h * 16, 16)]
                        cbuf[q, pl.ds(64 + h * 16, 16)] = \
                            ring[r1, pl.ds(poff + h * 16, 16)]
                    return c
                lax.fori_loop(0, G * H, comp, 0)

            def blk_body(blk, carry):
                pltpu.async_copy(
                    pidx.at[tok_v.at[pl.ds(blk * HOP1, HOP1)]],
                    neighs_v, sem_h).wait()
                row_blk = (base + blk * HOP1) * H

                def pair_body(p, carry2):
                    toff = p * 2 * G
                    cps_a = fire(ring0, toff)
                    cps_b = fire(ring1, toff + G)
                    for cp in cps_a:
                        cp.wait()
                    compact(ring0, cbuf0)
                    w0 = pltpu.async_copy(
                        cbuf0, out_hbm.at[pl.ds(row_blk + toff * H, G * H)],
                        sem_w0)
                    for cp in cps_b:
                        cp.wait()
                    compact(ring1, cbuf1)
                    w1 = pltpu.async_copy(
                        cbuf1,
                        out_hbm.at[pl.ds(row_blk + (toff + G) * H, G * H)],
                        sem_w1)
                    w0.wait()
                    w1.wait()
                    return carry2

                lax.fori_loop(0, HOP1 // (2 * G), pair_body, 0)
                return carry

            lax.fori_loop(0, TPW // HOP1, blk_body, 0)

        do_side(uids, 0, 0, gu_out)
        do_side(iids, K, 64, gi_out)

    return gather_k


def _tc_body(gu_ref, gi_ref, w1_ref, b1_ref, w2_ref, b2_ref, w3_ref, b3_ref,
             out_ref, *, T, K):
    H = K // 2
    R = T * H
    f32 = jnp.float32
    bf16 = jnp.bfloat16
    dn = (((2,), (1,)), ((0,), (0,)))

    def scored(g_ref):
        g3 = g_ref[...].reshape(T, H, 128)
        s_lo = g3[:, :, 0:K].astype(bf16)
        e_lo = g3[:, :, K:2 * K].astype(bf16)
        s_hi = g3[:, :, 2 * K:3 * K].astype(bf16)
        e_hi = g3[:, :, 3 * K:].astype(bf16)
        sc_lo = (lax.dot_general(s_lo[:, :, :H], e_lo, dn,
                                 preferred_element_type=f32) +
                 lax.dot_general(s_lo[:, :, H:], e_hi, dn,
                                 preferred_element_type=f32))
        sc_hi = (lax.dot_general(s_hi[:, :, :H], e_lo, dn,
                                 preferred_element_type=f32) +
                 lax.dot_general(s_hi[:, :, H:], e_hi, dn,
                                 preferred_element_type=f32))
        return sc_lo.reshape(R, K), sc_hi.reshape(R, K)

    su_lo, su_hi = scored(gu_ref)
    si_lo, si_hi = scored(gi_ref)

    def head(su, si):
        cat = jnp.concatenate([su, si], axis=1)
        h = jnp.dot(cat, w1_ref[...], preferred_element_type=f32) + b1_ref[...]
        h = jnp.maximum(h, 0.0)
        h = jnp.dot(h, w2_ref[...], preferred_element_type=f32) + b2_ref[...]
        h = jnp.maximum(h, 0.0)
        o = jnp.dot(h, w3_ref[...], preferred_element_type=f32) + b3_ref[...]
        sg = 1.0 / (1.0 + jnp.exp(-o))
        return jnp.mean(sg.reshape(T, H), axis=1)

    out_ref[...] = 0.5 * (head(su_lo, si_lo) + head(su_hi, si_hi))


def _dense(gu, gi, W1, b1, W2, b2, W3, b3, *, B, K, T):
    R = T * (K // 2)
    grid = B // T
    return pl.pallas_call(
        functools.partial(_tc_body, T=T, K=K),
        grid=(grid,),
        in_specs=[
            pl.BlockSpec((R, 128), lambda i: (i, 0)),
            pl.BlockSpec((R, 128), lambda i: (i, 0)),
            pl.BlockSpec(W1.shape, lambda i: (0, 0)),
            pl.BlockSpec(b1.shape, lambda i: (0, 0)),
            pl.BlockSpec(W2.shape, lambda i: (0, 0)),
            pl.BlockSpec(b2.shape, lambda i: (0, 0)),
            pl.BlockSpec(W3.shape, lambda i: (0, 0)),
            pl.BlockSpec(b3.shape, lambda i: (0, 0)),
        ],
        out_specs=pl.BlockSpec((T,), lambda i: (i,)),
        out_shape=jax.ShapeDtypeStruct((B,), jnp.float32),
    )(gu, gi, W1, b1, W2, b2, W3, b3)


def kernel(user_idxs, item_idxs, user_idx_tensor, user_scr_tensor,
           item_idx_tensor, item_scr_tensor, user_emb_table, item_emb_table,
           W1, b1, W2, b2, W3, b3):
    B = user_idxs.shape[0]
    N, K = user_idx_tensor.shape
    packed_tab = jnp.concatenate(
        [user_scr_tensor, user_emb_table,
         item_scr_tensor, item_emb_table], axis=1)
    packed_idx = jnp.concatenate(
        [user_idx_tensor, item_idx_tensor,
         jnp.zeros((N, 128 - 2 * K), jnp.int32)], axis=1)
    CH = 4
    Bc = B // CH
    gather_k = _make_gather(Bc, K)
    outs = []
    for c in range(CH):
        sl = slice(c * Bc, (c + 1) * Bc)
        gu, gi = gather_k(user_idxs[sl], item_idxs[sl], packed_idx, packed_tab)
        outs.append(_dense(gu, gi, W1, b1.reshape(1, -1), W2, b2.reshape(1, -1),
                           W3, b3.reshape(1, 1), B=Bc, K=K, T=256))
    return jnp.concatenate(outs)


# speedup vs baseline: 1.2180x; 1.2180x over previous
import functools

import jax
import jax.numpy as jnp
from jax import lax
from jax.experimental import pallas as pl
from jax.experimental.pallas import tpu as pltpu
from jax.experimental.pallas import tpu_sc as plsc

NW = 32
HOP1 = 64
G = 8


def _make_gather(B, K):
    TPW = B // NW
    mesh = plsc.VectorSubcoreMesh(core_axis_name="c", subcore_axis_name="s")

    @functools.partial(
        pl.kernel,
        mesh=mesh,
        out_type=[
            jax.ShapeDtypeStruct((B * K, 128), jnp.float32),
            jax.ShapeDtypeStruct((B * K, 128), jnp.float32),
        ],
        scratch_types=[
            pltpu.VMEM((TPW,), jnp.int32),
            pltpu.VMEM((HOP1, 128), jnp.int32),
            pltpu.VMEM((G * K, 128), jnp.float32),
            pltpu.VMEM((G * K, 128), jnp.float32),
            pltpu.SemaphoreType.DMA,
            pltpu.SemaphoreType.DMA,
            pltpu.SemaphoreType.DMA,
            pltpu.SemaphoreType.DMA,
        ],
    )
    def gather_k(uids, iids, pidx, ptab, gu_out, gi_out,
                 tok_v, neighs_v, ring0, ring1, sem_h, sem_g, sem_w0, sem_w1):
        wid = lax.axis_index("s") * 2 + lax.axis_index("c")
        base = wid * TPW
        rings = (ring0, ring1)
        wsems = (sem_w0, sem_w1)

        def do_side(ids_hbm, col_off, out_hbm):
            pltpu.sync_copy(ids_hbm.at[pl.ds(base, TPW)], tok_v)

            def blk_body(blk, carry):
                pltpu.async_copy(
                    pidx.at[tok_v.at[pl.ds(blk * HOP1, HOP1)]],
                    neighs_v, sem_h).wait()
                writes = {}
                for g in range((HOP1 + G - 1) // G):
                    ring = rings[g % 2]
                    cps = [
                        pltpu.async_copy(
                            ptab.at[neighs_v.at[g * G + t, pl.ds(col_off, K)]],
                            ring.at[pl.ds(t * K, K)], sem_g)
                        for t in range(G)
                    ]
                    for cp in cps:
                        cp.wait()
                    if (g % 2) in writes:
                        writes[g % 2].wait()
                    row0 = (base + blk * HOP1 + g * G) * K
                    writes[g % 2] = pltpu.async_copy(
                        ring, out_hbm.at[pl.ds(row0, G * K)], wsems[g % 2])
                for w in writes.values():
                    w.wait()
                return carry

            lax.fori_loop(0, TPW // HOP1, blk_body, 0)

        do_side(uids, 0, gu_out)
        do_side(iids, K, gi_out)

    return gather_k


def _tc_body(gu_ref, gi_ref, w1_ref, b1_ref, w2_ref, b2_ref, w3_ref, b3_ref,
             out_ref, *, T, K):
    R = T * K
    P = 256 // K
    f32 = jnp.float32

    def scored(g_ref, c0):
        g = g_ref[...]
        s = g[:, c0:c0 + K].astype(jnp.bfloat16)
        e = g[:, c0 + K:c0 + 2 * K].astype(jnp.bfloat16)
        s3 = s.reshape(T, K, K)
        e3 = e.reshape(T, K, K)
        sc = lax.dot_general(s3, e3, (((2,), (1,)), ((0,), (0,))),
                             preferred_element_type=f32)
        return sc.reshape(R, K)

    su = scored(gu_ref, 0)
    si = scored(gi_ref, 2 * K)
    cat = jnp.concatenate([su, si], axis=1)
    h = jnp.dot(cat, w1_ref[...], preferred_element_type=f32) + b1_ref[...]
    h = jnp.maximum(h, 0.0)
    h = jnp.dot(h, w2_ref[...], preferred_element_type=f32) + b2_ref[...]
    h = jnp.maximum(h, 0.0)
    o = jnp.dot(h, w3_ref[...], preferred_element_type=f32) + b3_ref[...]
    sg = 1.0 / (1.0 + jnp.exp(-o))
    out_ref[...] = jnp.mean(sg.reshape(T, K), axis=1)


def _dense(gu, gi, W1, b1, W2, b2, W3, b3, *, B, K, T):
    R = T * K
    F = gu.shape[1]
    grid = B // T
    return pl.pallas_call(
        functools.partial(_tc_body, T=T, K=K),
        grid=(grid,),
        in_specs=[
            pl.BlockSpec((R, F), lambda i: (i, 0)),
            pl.BlockSpec((R, F), lambda i: (i, 0)),
            pl.BlockSpec(W1.shape, lambda i: (0, 0)),
            pl.BlockSpec(b1.shape, lambda i: (0, 0)),
            pl.BlockSpec(W2.shape, lambda i: (0, 0)),
            pl.BlockSpec(b2.shape, lambda i: (0, 0)),
            pl.BlockSpec(W3.shape, lambda i: (0, 0)),
            pl.BlockSpec(b3.shape, lambda i: (0, 0)),
        ],
        out_specs=pl.BlockSpec((T,), lambda i: (i,)),
        out_shape=jax.ShapeDtypeStruct((B,), jnp.float32),
    )(gu, gi, W1, b1, W2, b2, W3, b3)


def kernel(user_idxs, item_idxs, user_idx_tensor, user_scr_tensor,
           item_idx_tensor, item_scr_tensor, user_emb_table, item_emb_table,
           W1, b1, W2, b2, W3, b3):
    B = user_idxs.shape[0]
    N, K = user_idx_tensor.shape
    packed_tab = jnp.concatenate(
        [user_scr_tensor, user_emb_table,
         item_scr_tensor, item_emb_table], axis=1)
    packed_idx = jnp.concatenate(
        [user_idx_tensor, item_idx_tensor,
         jnp.zeros((N, 128 - 2 * K), jnp.int32)], axis=1)
    CH = 8
    Bc = B // CH
    gather_k = _make_gather(Bc, K)
    outs = []
    for c in range(CH):
        sl = slice(c * Bc, (c + 1) * Bc)
        gu, gi = gather_k(user_idxs[sl], item_idxs[sl], packed_idx, packed_tab)
        outs.append(_dense(gu, gi, W1, b1.reshape(1, -1), W2, b2.reshape(1, -1),
                           W3, b3.reshape(1, 1), B=Bc, K=K, T=256))
    return jnp.concatenate(outs)


# speedup vs baseline: 1.2489x; 1.0254x over previous
import functools

import jax
import jax.numpy as jnp
from jax import lax
from jax.experimental import pallas as pl
from jax.experimental.pallas import tpu as pltpu
from jax.experimental.pallas import tpu_sc as plsc

NW = 32
HOP1 = 128
G = 8


def _make_gather(B, K):
    TPW = B // NW
    mesh = plsc.VectorSubcoreMesh(core_axis_name="c", subcore_axis_name="s")

    @functools.partial(
        pl.kernel,
        mesh=mesh,
        out_type=[
            jax.ShapeDtypeStruct((B * K, 128), jnp.float32),
            jax.ShapeDtypeStruct((B * K, 128), jnp.float32),
        ],
        scratch_types=[
            pltpu.VMEM((TPW,), jnp.int32),
            pltpu.VMEM((HOP1, 128), jnp.int32),
            pltpu.VMEM((G * K, 128), jnp.float32),
            pltpu.VMEM((G * K, 128), jnp.float32),
            pltpu.SemaphoreType.DMA,
            pltpu.SemaphoreType.DMA,
            pltpu.SemaphoreType.DMA,
            pltpu.SemaphoreType.DMA,
        ],
    )
    def gather_k(uids, iids, pidx, ptab, gu_out, gi_out,
                 tok_v, neighs_v, ring0, ring1, sem_h, sem_g, sem_w0, sem_w1):
        wid = lax.axis_index("s") * 2 + lax.axis_index("c")
        base = wid * TPW
        rings = (ring0, ring1)
        wsems = (sem_w0, sem_w1)

        def do_side(ids_hbm, col_off, out_hbm):
            pltpu.sync_copy(ids_hbm.at[pl.ds(base, TPW)], tok_v)

            def blk_body(blk, carry):
                pltpu.async_copy(
                    pidx.at[tok_v.at[pl.ds(blk * HOP1, HOP1)]],
                    neighs_v, sem_h).wait()
                writes = {}
                for g in range((HOP1 + G - 1) // G):
                    ring = rings[g % 2]
                    cps = [
                        pltpu.async_copy(
                            ptab.at[neighs_v.at[g * G + t, pl.ds(col_off, K)]],
                            ring.at[pl.ds(t * K, K)], sem_g)
                        for t in range(G)
                    ]
                    for cp in cps:
                        cp.wait()
                    if (g % 2) in writes:
                        writes[g % 2].wait()
                    row0 = (base + blk * HOP1 + g * G) * K
                    writes[g % 2] = pltpu.async_copy(
                        ring, out_hbm.at[pl.ds(row0, G * K)], wsems[g % 2])
                for w in writes.values():
                    w.wait()
                return carry

            lax.fori_loop(0, TPW // HOP1, blk_body, 0)

        do_side(uids, 0, gu_out)
        do_side(iids, K, gi_out)

    return gather_k


def _tc_body(gu_ref, gi_ref, w1_ref, b1_ref, w2_ref, b2_ref, w3_ref, b3_ref,
             out_ref, *, T, K):
    R = T * K
    f32 = jnp.float32

    def scored(g_ref, c0):
        g = g_ref[...]
        s = g[:, c0:c0 + K].astype(jnp.bfloat16)
        e = g[:, c0 + K:c0 + 2 * K].astype(jnp.bfloat16)
        s3 = s.reshape(T, K, K)
        e3 = e.reshape(T, K, K)
        sc = lax.dot_general(s3, e3, (((2,), (1,)), ((0,), (0,))),
                             preferred_element_type=f32)
        return sc.reshape(R, K)

    su = scored(gu_ref, 0)
    si = scored(gi_ref, 2 * K)
    cat = jnp.concatenate([su, si], axis=1)
    h = jnp.dot(cat, w1_ref[...], preferred_element_type=f32) + b1_ref[...]
    h = jnp.maximum(h, 0.0)
    h = jnp.dot(h, w2_ref[...], preferred_element_type=f32) + b2_ref[...]
    h = jnp.maximum(h, 0.0)
    o = jnp.dot(h, w3_ref[...], preferred_element_type=f32) + b3_ref[...]
    sg = 1.0 / (1.0 + jnp.exp(-o))
    out_ref[...] = jnp.mean(sg.reshape(T, K), axis=1)


def _dense(gu, gi, W1, b1, W2, b2, W3, b3, *, B, K, T):
    R = T * K
    F = gu.shape[1]
    grid = B // T
    return pl.pallas_call(
        functools.partial(_tc_body, T=T, K=K),
        grid=(grid,),
        in_specs=[
            pl.BlockSpec((R, F), lambda i: (i, 0)),
            pl.BlockSpec((R, F), lambda i: (i, 0)),
            pl.BlockSpec(W1.shape, lambda i: (0, 0)),
            pl.BlockSpec(b1.shape, lambda i: (0, 0)),
            pl.BlockSpec(W2.shape, lambda i: (0, 0)),
            pl.BlockSpec(b2.shape, lambda i: (0, 0)),
            pl.BlockSpec(W3.shape, lambda i: (0, 0)),
            pl.BlockSpec(b3.shape, lambda i: (0, 0)),
        ],
        out_specs=pl.BlockSpec((T,), lambda i: (i,)),
        out_shape=jax.ShapeDtypeStruct((B,), jnp.float32),
    )(gu, gi, W1, b1, W2, b2, W3, b3)


def kernel(user_idxs, item_idxs, user_idx_tensor, user_scr_tensor,
           item_idx_tensor, item_scr_tensor, user_emb_table, item_emb_table,
           W1, b1, W2, b2, W3, b3):
    B = user_idxs.shape[0]
    N, K = user_idx_tensor.shape
    packed_tab = jnp.concatenate(
        [user_scr_tensor, user_emb_table,
         item_scr_tensor, item_emb_table], axis=1)
    packed_idx = jnp.concatenate(
        [user_idx_tensor, item_idx_tensor,
         jnp.zeros((N, 128 - 2 * K), jnp.int32)], axis=1)
    CH = 4
    Bc = B // CH
    gather_k = _make_gather(Bc, K)
    outs = []
    for c in range(CH):
        sl = slice(c * Bc, (c + 1) * Bc)
        gu, gi = gather_k(user_idxs[sl], item_idxs[sl], packed_idx, packed_tab)
        outs.append(_dense(gu, gi, W1, b1.reshape(1, -1), W2, b2.reshape(1, -1),
                           W3, b3.reshape(1, 1), B=Bc, K=K, T=256))
    return jnp.concatenate(outs)
